# baseline (speedup 1.0000x reference)
import functools

import jax
import jax.numpy as jnp
from jax import lax
from jax.experimental import pallas as pl
from jax.experimental.pallas import tpu as pltpu
from jax.experimental.pallas import tpu_sc as plsc

V = 1000000
D = 64
B_TOK = 4096
S_TOK = 200
LANES = 16
SCALE = 8.0
K1_W = 1024
K3_H = 512


def _k1_tc(t_t):
    def body(x_ref, o_ref):
        x = x_ref[...]
        o_ref[...] = (
            x.reshape(D, K1_W // 2, 2).transpose(1, 2, 0).reshape(K1_W // 2, 128)
            * SCALE
        )

    grid = (V + K1_W - 1) // K1_W
    return pl.pallas_call(
        body,
        grid=(grid,),
        in_specs=[pl.BlockSpec((D, K1_W), lambda i: (0, i))],
        out_specs=pl.BlockSpec((K1_W // 2, 128), lambda i: (i, 0)),
        out_shape=jax.ShapeDtypeStruct((V // 2, 128), jnp.float32),
    )(t_t)


def _k3_tc(out2):
    nc = B_TOK // (2 * K3_H)

    def body(x_ref, o_ref):
        x = x_ref[...]
        o_ref[...] = x.reshape(K3_H, 2, D).transpose(2, 0, 1).reshape(
            1, D, 2 * K3_H)

    return pl.pallas_call(
        body,
        grid=(S_TOK, nc),
        in_specs=[pl.BlockSpec((K3_H, 128), lambda s, c: (s * nc + c, 0))],
        out_specs=pl.BlockSpec((1, D, 2 * K3_H), lambda s, c: (s, 0, c)),
        out_shape=jax.ShapeDtypeStruct((S_TOK, D, B_TOK), jnp.float32),
    )(out2)


def _make_k2():
    info = plsc.get_sparse_core_info()
    nc, ns = info.num_cores, info.num_subcores
    mesh = plsc.VectorSubcoreMesh(core_axis_name="c", subcore_axis_name="s")

    @functools.partial(
        pl.kernel,
        mesh=mesh,
        out_type=jax.ShapeDtypeStruct((S_TOK * B_TOK // 2, 128), jnp.float32),
        scratch_types=[
            pltpu.VMEM((S_TOK, 128), jnp.int32),
            pltpu.VMEM((S_TOK, 128), jnp.int32),
            pltpu.VMEM((128, 128), jnp.float32),
            pltpu.VMEM((128, 128), jnp.float32),
            pltpu.VMEM((64, 128), jnp.float32),
            pltpu.VMEM((64, 128), jnp.float32),
            pltpu.SemaphoreType.DMA,
            pltpu.SemaphoreType.DMA,
            pltpu.SemaphoreType.DMA,
            pltpu.SemaphoreType.DMA,
        ],
        compiler_params=pltpu.CompilerParams(use_tc_tiling_on_sc=False),
    )
    def k2(t2, sidx, out2, idx_v, idxh_v, g0, g1, v0, v1, gs0, gs1, ws0, ws1):
        w = lax.axis_index("s") * nc + lax.axis_index("c")
        pltpu.sync_copy(sidx.at[:, w], idx_v)

        @plsc.parallel_loop(0, S_TOK * 128 // LANES, unroll=4)
        def _(i):
            r = i >> 3
            c0 = (i & 7) * LANES
            idxh_v[r, pl.ds(c0, LANES)] = idx_v[r, pl.ds(c0, LANES)] >> 1

        def fire_gather(s, grow, gsem):
            pltpu.async_copy(t2.at[idxh_v.at[s]], grow, gsem)

        def wait_gather(grow, gsem):
            pltpu.make_async_copy(t2.at[idxh_v.at[0]], grow, gsem).wait()

        def out_slice(s):
            return out2.at[pl.ds(s * (B_TOK // 2) + 64 * w, 64)]

        def fire_write(s, vout, wsem):
            pltpu.async_copy(vout, out_slice(s), wsem)

        def wait_write(vout, wsem):
            pltpu.make_async_copy(vout, out_slice(0), wsem).wait()

        def compact(s, grow, vout):
            for t0 in range(0, 128, LANES):
                vv = idx_v[s, pl.ds(t0, LANES)]
                for l in range(LANES):
                    t = t0 + l
                    par = (vv[l] & 1) << 6
                    for k in range(D // LANES):
                        vout[t // 2, pl.ds((t % 2) * D + k * LANES, LANES)] = (
                            grow[t, pl.ds(par + k * LANES, LANES)])

        fire_gather(0, g0, gs0)

        @pl.loop(0, S_TOK, step=2)
        def _(a):
            @pl.when(a > 0)
            def _():
                wait_write(v1, ws1)

            fire_gather(a + 1, g1, gs1)
            wait_gather(g0, gs0)
            compact(a, g0, v0)
            fire_write(a, v0, ws0)
            wait_write(v0, ws0)

            @pl.when(a + 2 < S_TOK)
            def _():
                fire_gather(a + 2, g0, gs0)

            wait_gather(g1, gs1)
            compact(a + 1, g1, v1)
            fire_write(a + 1, v1, ws1)

        wait_write(v1, ws1)

    return k2


def kernel(src, table):
    t_t = jnp.transpose(table)
    t2 = _k1_tc(t_t)
    sidx = jnp.transpose(src).astype(jnp.int32).reshape(S_TOK, 32, 128)
    out2 = _make_k2()(t2, sidx)
    o3 = _k3_tc(out2)
    return jnp.transpose(o3, (2, 0, 1))


# speedup vs baseline: 13.0354x; 13.0354x over previous
import functools

import jax
import jax.numpy as jnp
from jax import lax
from jax.experimental import pallas as pl
from jax.experimental.pallas import tpu as pltpu
from jax.experimental.pallas import tpu_sc as plsc

V = 1000000
D = 64
B_TOK = 4096
S_TOK = 200
SCALE = 8.0
K1_W = 2048
K3_H = 512


def _k1_tc(t_t):
    def body(x_ref, o_ref):
        x = x_ref[...] * SCALE
        o_ref[...] = jnp.concatenate(
            [jnp.transpose(x), jnp.zeros((K1_W, D), jnp.float32)], axis=1)

    grid = (V + K1_W - 1) // K1_W
    return pl.pallas_call(
        body,
        grid=(grid,),
        in_specs=[pl.BlockSpec((D, K1_W), lambda i: (0, i))],
        out_specs=pl.BlockSpec((K1_W, 128), lambda i: (i, 0)),
        out_shape=jax.ShapeDtypeStruct((V, 128), jnp.float32),
    )(t_t)


def _k3_tc(out2):
    nc = B_TOK // (2 * K3_H)

    def body(x_ref, o_ref):
        x = x_ref[...]
        o_ref[...] = jnp.concatenate(
            [jnp.transpose(x[:, :D]), jnp.transpose(x[:, D:])], axis=1
        ).reshape(1, D, 2 * K3_H)

    return pl.pallas_call(
        body,
        grid=(S_TOK, nc),
        in_specs=[pl.BlockSpec((K3_H, 128), lambda s, c: (s * nc + c, 0))],
        out_specs=pl.BlockSpec((1, D, 2 * K3_H), lambda s, c: (s, 0, c)),
        out_shape=jax.ShapeDtypeStruct((S_TOK, D, B_TOK), jnp.float32),
    )(out2)


def _make_k2():
    info = plsc.get_sparse_core_info()
    nc, ns = info.num_cores, info.num_subcores
    mesh = plsc.VectorSubcoreMesh(core_axis_name="c", subcore_axis_name="s")
    half = B_TOK // 2

    @functools.partial(
        pl.kernel,
        mesh=mesh,
        out_type=jax.ShapeDtypeStruct((S_TOK * half, 128), jnp.float32),
        scratch_types=[
            pltpu.VMEM((S_TOK, 128), jnp.int32),
            pltpu.VMEM((128, 128), jnp.float32),
            pltpu.VMEM((128, 128), jnp.float32),
            pltpu.SemaphoreType.DMA,
            pltpu.SemaphoreType.DMA,
            pltpu.SemaphoreType.DMA,
            pltpu.SemaphoreType.DMA,
        ],
        compiler_params=pltpu.CompilerParams(use_tc_tiling_on_sc=False),
    )
    def k2(t2p, sidx, out2, idx_v, g0, g1, gs0, gs1, ws0, ws1):
        w = lax.axis_index("s") * nc + lax.axis_index("c")
        pltpu.sync_copy(sidx.at[:, w], idx_v)
        row0 = 512 * (w // 8) + 128 * (w % 4)
        col0 = D * ((w // 4) % 2)

        def fire_gather(s, grow, gsem):
            pltpu.async_copy(t2p.at[idx_v.at[s]], grow, gsem)

        def wait_gather(grow, gsem):
            pltpu.make_async_copy(t2p.at[idx_v.at[0]], grow, gsem).wait()

        def out_slice(s):
            return out2.at[pl.ds(s * half + row0, 128), pl.ds(col0, D)]

        def fire_write(s, grow, wsem):
            pltpu.async_copy(grow.at[:, pl.ds(0, D)], out_slice(s), wsem)

        def wait_write(grow, wsem):
            pltpu.make_async_copy(
                grow.at[:, pl.ds(0, D)], out_slice(0), wsem).wait()

        fire_gather(0, g0, gs0)

        @pl.loop(0, S_TOK, step=2)
        def _(a):
            @pl.when(a > 0)
            def _():
                wait_write(g1, ws1)

            fire_gather(a + 1, g1, gs1)
            wait_gather(g0, gs0)
            fire_write(a, g0, ws0)
            wait_write(g0, ws0)

            @pl.when(a + 2 < S_TOK)
            def _():
                fire_gather(a + 2, g0, gs0)

            wait_gather(g1, gs1)
            fire_write(a + 1, g1, ws1)

        wait_write(g1, ws1)

    return k2


def kernel(src, table):
    t_t = jnp.transpose(table)
    t2p = _k1_tc(t_t)
    sidx = jnp.transpose(src).astype(jnp.int32).reshape(S_TOK, 32, 128)
    out2 = _make_k2()(t2p, sidx)
    o3 = _k3_tc(out2)
    return jnp.transpose(o3, (2, 0, 1))
